# baseline (speedup 1.0000x reference)
import functools

import jax
import jax.numpy as jnp
from jax import lax
from jax.experimental import pallas as pl
from jax.experimental.pallas import tpu as pltpu
from jax.experimental.pallas import tpu_sc as plsc

S = 2048
D = 1024
H = 16
DH = 64
E = 8
DFF = 4096
C = 512
EC = E * C
SCALE = 1.0 / (DH ** 0.5)
EPS_LN = 1e-5

NC = 2
NS = 16
NL = 16
NW = NC * NS
SLOTS_PER_W = EC // NW
TOK_PER_W = S // NW
SRC_N = EC + 8


def _ln(x, g, b):
    mu = jnp.mean(x, axis=1, keepdims=True)
    var = jnp.mean((x - mu) ** 2, axis=1, keepdims=True)
    return (x - mu) * lax.rsqrt(var + EPS_LN) * g + b


def _qkv_kern(x_ref, w_ref, g_ref, b_ref, out_ref):
    h = _ln(x_ref[...], g_ref[...], b_ref[...])
    out_ref[...] = jnp.dot(h, w_ref[...], preferred_element_type=jnp.float32)


def _qkv_call(x2, wqkv, g, b):
    bm = 256
    return pl.pallas_call(
        _qkv_kern,
        grid=(S // bm, 3),
        in_specs=[
            pl.BlockSpec((bm, D), lambda i, n: (i, 0)),
            pl.BlockSpec((D, D), lambda i, n: (0, n)),
            pl.BlockSpec((1, D), lambda i, n: (0, 0)),
            pl.BlockSpec((1, D), lambda i, n: (0, 0)),
        ],
        out_specs=pl.BlockSpec((bm, D), lambda i, n: (i, n)),
        out_shape=jax.ShapeDtypeStruct((S, 3 * D), jnp.float32),
    )(x2, wqkv, g, b)


def _attn_kern(q_ref, k_ref, v_ref, o_ref):
    q = q_ref[0]
    k = k_ref[0]
    v = v_ref[0]
    s = lax.dot_general(q, k, (((1,), (1,)), ((), ())),
                        preferred_element_type=jnp.float32) * SCALE
    m = jnp.max(s, axis=1, keepdims=True)
    e = jnp.exp(s - m)
    p = e / jnp.sum(e, axis=1, keepdims=True)
    o_ref[0] = jnp.dot(p, v, preferred_element_type=jnp.float32)


def _attn_call(q4, k4, v4):
    bq = 256
    return pl.pallas_call(
        _attn_kern,
        grid=(H, S // bq),
        in_specs=[
            pl.BlockSpec((1, bq, DH), lambda h, i: (h, i, 0)),
            pl.BlockSpec((1, S, DH), lambda h, i: (h, 0, 0)),
            pl.BlockSpec((1, S, DH), lambda h, i: (h, 0, 0)),
        ],
        out_specs=pl.BlockSpec((1, bq, DH), lambda h, i: (h, i, 0)),
        out_shape=jax.ShapeDtypeStruct((H, S, DH), jnp.float32),
    )(q4, k4, v4)


def _post_kern(ctx_ref, x_ref, wo_ref, g_ref, b_ref, wg_ref,
               y_ref, tok_ref, logit_ref):
    y = x_ref[...] + jnp.dot(ctx_ref[...], wo_ref[...],
                             preferred_element_type=jnp.float32)
    y_ref[...] = y
    t = _ln(y, g_ref[...], b_ref[...])
    tok_ref[...] = t
    logit_ref[...] = jnp.dot(t, wg_ref[...], preferred_element_type=jnp.float32)


def _post_call(ctx2, x2, wo, g, b, wg):
    bm = 256
    return pl.pallas_call(
        _post_kern,
        grid=(S // bm,),
        in_specs=[
            pl.BlockSpec((bm, D), lambda i: (i, 0)),
            pl.BlockSpec((bm, D), lambda i: (i, 0)),
            pl.BlockSpec((D, D), lambda i: (0, 0)),
            pl.BlockSpec((1, D), lambda i: (0, 0)),
            pl.BlockSpec((1, D), lambda i: (0, 0)),
            pl.BlockSpec((D, E), lambda i: (0, 0)),
        ],
        out_specs=[
            pl.BlockSpec((bm, D), lambda i: (i, 0)),
            pl.BlockSpec((bm, D), lambda i: (i, 0)),
            pl.BlockSpec((bm, E), lambda i: (i, 0)),
        ],
        out_shape=[
            jax.ShapeDtypeStruct((S, D), jnp.float32),
            jax.ShapeDtypeStruct((S, D), jnp.float32),
            jax.ShapeDtypeStruct((S, E), jnp.float32),
        ],
    )(ctx2, x2, wo, g, b, wg)


def _gate_kern(logit_ref, sd1_ref, sd2_ref, sc1_ref, sc2_ref, g1_ref, g2_ref):
    logits = logit_ref[...]
    m = jnp.max(logits, axis=1, keepdims=True)
    ex = jnp.exp(logits - m)
    gates = ex / jnp.sum(ex, axis=1, keepdims=True)
    col = lax.broadcasted_iota(jnp.int32, (S, E), 1)
    gmax = jnp.max(gates, axis=1, keepdims=True)
    idx1 = jnp.min(jnp.where(gates == gmax, col, E), axis=1)
    mask1 = col == idx1[:, None]
    lx = jnp.where(mask1, -jnp.inf, logits)
    lmax2 = jnp.max(lx, axis=1, keepdims=True)
    idx2 = jnp.min(jnp.where(lx == lmax2, col, E), axis=1)
    mask2 = col == idx2[:, None]
    m1f = mask1.astype(jnp.float32)
    m2f = mask2.astype(jnp.float32)

    tri = (lax.broadcasted_iota(jnp.int32, (S, S), 0)
           >= lax.broadcasted_iota(jnp.int32, (S, S), 1)).astype(jnp.bfloat16)
    cs1 = jnp.dot(tri, m1f.astype(jnp.bfloat16),
                  preferred_element_type=jnp.float32)
    cs2 = jnp.dot(tri, m2f.astype(jnp.bfloat16),
                  preferred_element_type=jnp.float32)
    loc1 = cs1 - 1.0
    loc2 = cs2 - 1.0 + jnp.sum(m1f, axis=0, keepdims=True)
    keep1 = m1f * (loc1 < C).astype(jnp.float32)
    keep2 = m2f * (loc2 < C).astype(jnp.float32)
    loc1s = jnp.sum(loc1 * keep1, axis=1).astype(jnp.int32)
    loc2s = jnp.sum(loc2 * keep2, axis=1).astype(jnp.int32)
    g1s = jnp.sum(gates * keep1, axis=1)
    g2s = jnp.sum(gates * keep2, axis=1)
    denom = jnp.maximum(g1s + g2s, jnp.float32(1.1920929e-07))
    g1n = g1s / denom
    g2n = g2s / denom
    k1 = jnp.sum(keep1, axis=1) > 0.0
    k2 = jnp.sum(keep2, axis=1) > 0.0
    slot1 = idx1 * C + loc1s
    slot2 = idx2 * C + loc2s
    sd1_ref[...] = jnp.where(k1, slot1, EC)
    sd2_ref[...] = jnp.where(k2, slot2, EC)
    sc1_ref[...] = jnp.where(k1, slot1, 0)
    sc2_ref[...] = jnp.where(k2, slot2, 0)
    zl = jnp.zeros((S, NL), jnp.float32)
    g1_ref[...] = jnp.where(k1, g1n, 0.0)[:, None] + zl
    g2_ref[...] = jnp.where(k2, g2n, 0.0)[:, None] + zl


def _gate_call(logits):
    return pl.pallas_call(
        _gate_kern,
        out_shape=[
            jax.ShapeDtypeStruct((S,), jnp.int32),
            jax.ShapeDtypeStruct((S,), jnp.int32),
            jax.ShapeDtypeStruct((S,), jnp.int32),
            jax.ShapeDtypeStruct((S,), jnp.int32),
            jax.ShapeDtypeStruct((S, NL), jnp.float32),
            jax.ShapeDtypeStruct((S, NL), jnp.float32),
        ],
    )(logits)


def _ffn_kern(x_ref, w1_ref, b1_ref, w2_ref, b2_ref, o_ref):
    f = pl.program_id(1)
    xb = x_ref[0].astype(jnp.bfloat16)
    h1 = jnp.dot(xb, w1_ref[0].astype(jnp.bfloat16),
                 preferred_element_type=jnp.float32)
    h1 = jax.nn.gelu(h1 + b1_ref[0])
    part = jnp.dot(h1.astype(jnp.bfloat16), w2_ref[0].astype(jnp.bfloat16),
                   preferred_element_type=jnp.float32)

    @pl.when(f == 0)
    def _():
        o_ref[0] = part + b2_ref[0]

    @pl.when(f > 0)
    def _():
        o_ref[0] += part


def _ffn_call(disp3, w1, b1, w2, b2):
    bf = 512
    return pl.pallas_call(
        _ffn_kern,
        grid=(E, DFF // bf),
        in_specs=[
            pl.BlockSpec((1, C, D), lambda e, f: (e, 0, 0)),
            pl.BlockSpec((1, D, bf), lambda e, f: (e, 0, f)),
            pl.BlockSpec((1, 1, bf), lambda e, f: (e, 0, f)),
            pl.BlockSpec((1, bf, D), lambda e, f: (e, f, 0)),
            pl.BlockSpec((1, 1, D), lambda e, f: (e, 0, 0)),
        ],
        out_specs=pl.BlockSpec((1, C, D), lambda e, f: (e, 0, 0)),
        out_shape=jax.ShapeDtypeStruct((E, C, D), jnp.float32),
    )(disp3, w1, b1.reshape(E, 1, DFF), w2, b2.reshape(E, 1, D))


@functools.cache
def _make_dispatch():
  mesh = plsc.VectorSubcoreMesh(core_axis_name="c", subcore_axis_name="s")

  @functools.partial(
      pl.kernel,
      mesh=mesh,
      out_type=jax.ShapeDtypeStruct((EC, D), jnp.float32),
      compiler_params=pltpu.CompilerParams(needs_layout_passes=False),
      scratch_types=[
          pltpu.VMEM((SRC_N,), jnp.int32),
          pltpu.VMEM((S,), jnp.int32),
          pltpu.VMEM((S,), jnp.int32),
          pltpu.VMEM((64, D), jnp.float32),
          pltpu.SemaphoreType.DMA,
      ],
  )
  def _dispatch_sc(sd1_hbm, sd2_hbm, tokpad_hbm, disp_hbm,
                   src_v, s1_v, s2_v, rows_v, sem):
    wid = lax.axis_index("s") * NC + lax.axis_index("c")
    pltpu.sync_copy(sd1_hbm, s1_v)
    pltpu.sync_copy(sd2_hbm, s2_v)

    fill = jnp.full((NL,), S, jnp.int32)

    def init_body(i, c):
      src_v[pl.ds(i * NL, NL)] = fill
      return c

    lax.fori_loop(0, SRC_N // NL, init_body, 0)

    def scat_body(i, c):
      ids = lax.broadcasted_iota(jnp.int32, (NL,), 0) + i * NL
      plsc.store_scatter(src_v, [s1_v[pl.ds(i * NL, NL)]], ids)
      plsc.store_scatter(src_v, [s2_v[pl.ds(i * NL, NL)]], ids)
      return c

    lax.fori_loop(0, S // NL, scat_body, 0)

    base = wid * SLOTS_PER_W

    def gath_body(j, c):
      off = base + j * 64
      pltpu.async_copy(tokpad_hbm.at[src_v.at[pl.ds(off, 64)]],
                       rows_v, sem).wait()
      pltpu.sync_copy(rows_v, disp_hbm.at[pl.ds(off, 64)])
      return c

    lax.fori_loop(0, SLOTS_PER_W // 64, gath_body, 0)

  return _dispatch_sc


@functools.cache
def _make_combine():
  mesh = plsc.VectorSubcoreMesh(core_axis_name="c", subcore_axis_name="s")

  @functools.partial(
      pl.kernel,
      mesh=mesh,
      out_type=jax.ShapeDtypeStruct((S, D), jnp.float32),
      scratch_types=[
          pltpu.VMEM((TOK_PER_W,), jnp.int32),
          pltpu.VMEM((TOK_PER_W,), jnp.int32),
          pltpu.VMEM((TOK_PER_W, NL), jnp.float32),
          pltpu.VMEM((TOK_PER_W, NL), jnp.float32),
          pltpu.VMEM((32, D), jnp.float32),
          pltpu.VMEM((32, D), jnp.float32),
          pltpu.VMEM((32, D), jnp.float32),
          pltpu.SemaphoreType.DMA,
          pltpu.SemaphoreType.DMA,
      ],
  )
  def _combine_sc(h2_hbm, y_hbm, sc1_hbm, sc2_hbm, g1_hbm, g2_hbm, out_hbm,
                  i1_v, i2_v, g1_v, g2_v, r1_v, r2_v, y_v, sem1, sem2):
    wid = lax.axis_index("s") * NC + lax.axis_index("c")
    tbase = wid * TOK_PER_W
    pltpu.sync_copy(sc1_hbm.at[pl.ds(tbase, TOK_PER_W)], i1_v)
    pltpu.sync_copy(sc2_hbm.at[pl.ds(tbase, TOK_PER_W)], i2_v)
    pltpu.sync_copy(g1_hbm.at[pl.ds(tbase, TOK_PER_W)], g1_v)
    pltpu.sync_copy(g2_hbm.at[pl.ds(tbase, TOK_PER_W)], g2_v)

    for half in range(TOK_PER_W // 32):
      t0 = tbase + half * 32
      cp1 = pltpu.async_copy(h2_hbm.at[i1_v.at[pl.ds(half * 32, 32)]],
                             r1_v, sem1)
      cp2 = pltpu.async_copy(h2_hbm.at[i2_v.at[pl.ds(half * 32, 32)]],
                             r2_v, sem2)
      pltpu.sync_copy(y_hbm.at[pl.ds(t0, 32)], y_v)
      cp1.wait()
      cp2.wait()

      def tok_body(t, c):
        a = g1_v[half * 32 + t, :]
        b = g2_v[half * 32 + t, :]

        def j_body(j, cc):
          sl = pl.ds(j * NL, NL)
          y_v[t, sl] = y_v[t, sl] + a * r1_v[t, sl] + b * r2_v[t, sl]
          return cc

        lax.fori_loop(0, D // NL, j_body, 0)
        return c

      lax.fori_loop(0, 32, tok_body, 0)
      pltpu.sync_copy(y_v, out_hbm.at[pl.ds(t0, 32)])

  return _combine_sc


def kernel(x, wq, wk, wv, wo, ln1_g, ln1_b, ln2_g, ln2_b, wg, w1, b1, w2, b2):
    x2 = x.reshape(S, D)
    wqkv = jnp.concatenate([wq, wk, wv], axis=1)
    qkv = _qkv_call(x2, wqkv, ln1_g.reshape(1, D), ln1_b.reshape(1, D))
    q4 = qkv[:, :D].reshape(S, H, DH).transpose(1, 0, 2)
    k4 = qkv[:, D:2 * D].reshape(S, H, DH).transpose(1, 0, 2)
    v4 = qkv[:, 2 * D:].reshape(S, H, DH).transpose(1, 0, 2)
    ctx4 = _attn_call(q4, k4, v4)
    ctx2 = ctx4.transpose(1, 0, 2).reshape(S, D)
    y, tok, logits = _post_call(ctx2, x2, wo, ln2_g.reshape(1, D),
                                ln2_b.reshape(1, D), wg)
    sd1, sd2, sc1, sc2, g1, g2 = _gate_call(logits)
    tokpad = jnp.concatenate([tok, jnp.zeros((8, D), jnp.float32)], axis=0)
    disp = _make_dispatch()(sd1, sd2, tokpad)
    h2 = _ffn_call(disp.reshape(E, C, D), w1, b1, w2, b2)
    out = _make_combine()(h2.reshape(EC, D), y, sc1, sc2, g1, g2)
    return out.reshape(1, S, D)


# speedup vs baseline: 1.0719x; 1.0719x over previous
import functools

import jax
import jax.numpy as jnp
from jax import lax
from jax.experimental import pallas as pl
from jax.experimental.pallas import tpu as pltpu
from jax.experimental.pallas import tpu_sc as plsc

S = 2048
D = 1024
H = 16
DH = 64
E = 8
DFF = 4096
C = 512
EC = E * C
SCALE = 1.0 / (DH ** 0.5)
EPS_LN = 1e-5

NC = 2
NS = 16
NL = 16
NW = NC * NS
SLOTS_PER_W = EC // NW
TOK_PER_W = S // NW
SRC_N = EC + 8


def _ln(x, g, b):
    mu = jnp.mean(x, axis=1, keepdims=True)
    var = jnp.mean((x - mu) ** 2, axis=1, keepdims=True)
    return (x - mu) * lax.rsqrt(var + EPS_LN) * g + b


def _qkv_kern(x_ref, wq_ref, wk_ref, wv_ref, g_ref, b_ref, out_ref):
    h = _ln(x_ref[...], g_ref[...], b_ref[...])
    out_ref[:, :D] = jnp.dot(h, wq_ref[...], preferred_element_type=jnp.float32)
    out_ref[:, D:2 * D] = jnp.dot(h, wk_ref[...],
                                  preferred_element_type=jnp.float32)
    out_ref[:, 2 * D:] = jnp.dot(h, wv_ref[...],
                                 preferred_element_type=jnp.float32)


def _qkv_call(x2, wq, wk, wv, g, b):
    bm = 256
    wspec = pl.BlockSpec((D, D), lambda i: (0, 0))
    return pl.pallas_call(
        _qkv_kern,
        grid=(S // bm,),
        in_specs=[
            pl.BlockSpec((bm, D), lambda i: (i, 0)),
            wspec, wspec, wspec,
            pl.BlockSpec((1, D), lambda i: (0, 0)),
            pl.BlockSpec((1, D), lambda i: (0, 0)),
        ],
        out_specs=pl.BlockSpec((bm, 3 * D), lambda i: (i, 0)),
        out_shape=jax.ShapeDtypeStruct((S, 3 * D), jnp.float32),
    )(x2, wq, wk, wv, g, b)


def _attn_kern(q_ref, k_ref, v_ref, o_ref):
    q = q_ref[0]
    k = k_ref[0]
    v = v_ref[0]
    s = lax.dot_general(q, k, (((1,), (1,)), ((), ())),
                        preferred_element_type=jnp.float32) * SCALE
    m = jnp.max(s, axis=1, keepdims=True)
    e = jnp.exp(s - m)
    p = e / jnp.sum(e, axis=1, keepdims=True)
    o_ref[0] = jnp.dot(p, v, preferred_element_type=jnp.float32)


def _attn_call(q4, k4, v4):
    bq = 256
    return pl.pallas_call(
        _attn_kern,
        grid=(H, S // bq),
        in_specs=[
            pl.BlockSpec((1, bq, DH), lambda h, i: (h, i, 0)),
            pl.BlockSpec((1, S, DH), lambda h, i: (h, 0, 0)),
            pl.BlockSpec((1, S, DH), lambda h, i: (h, 0, 0)),
        ],
        out_specs=pl.BlockSpec((1, bq, DH), lambda h, i: (h, i, 0)),
        out_shape=jax.ShapeDtypeStruct((H, S, DH), jnp.float32),
    )(q4, k4, v4)


def _post_kern(ctx_ref, x_ref, wo_ref, g_ref, b_ref, wg_ref,
               y_ref, tok_ref, logit_ref):
    i = pl.program_id(0)
    y = x_ref[...] + jnp.dot(ctx_ref[...], wo_ref[...],
                             preferred_element_type=jnp.float32)
    y_ref[...] = y
    t = _ln(y, g_ref[...], b_ref[...])
    logit_ref[...] = jnp.dot(t, wg_ref[...], preferred_element_type=jnp.float32)
    tok_ref[...] = jnp.where(i == S // t.shape[0], jnp.zeros_like(t), t)


def _post_call(ctx2, x2, wo, g, b, wg):
    bm = 256
    nb = S // bm
    clamp = lambda i: (jnp.minimum(i, nb - 1), 0)
    return pl.pallas_call(
        _post_kern,
        grid=(nb + 1,),
        in_specs=[
            pl.BlockSpec((bm, D), clamp),
            pl.BlockSpec((bm, D), clamp),
            pl.BlockSpec((D, D), lambda i: (0, 0)),
            pl.BlockSpec((1, D), lambda i: (0, 0)),
            pl.BlockSpec((1, D), lambda i: (0, 0)),
            pl.BlockSpec((D, E), lambda i: (0, 0)),
        ],
        out_specs=[
            pl.BlockSpec((bm, D), clamp),
            pl.BlockSpec((bm, D), lambda i: (i, 0)),
            pl.BlockSpec((bm, E), clamp),
        ],
        out_shape=[
            jax.ShapeDtypeStruct((S, D), jnp.float32),
            jax.ShapeDtypeStruct((S + bm, D), jnp.float32),
            jax.ShapeDtypeStruct((S, E), jnp.float32),
        ],
    )(ctx2, x2, wo, g, b, wg)


def _gate_kern(logit_ref, sd1_ref, sd2_ref, sc1_ref, sc2_ref, g1_ref, g2_ref):
    logits = logit_ref[...]
    m = jnp.max(logits, axis=1, keepdims=True)
    ex = jnp.exp(logits - m)
    gates = ex / jnp.sum(ex, axis=1, keepdims=True)
    col = lax.broadcasted_iota(jnp.int32, (S, E), 1)
    gmax = jnp.max(gates, axis=1, keepdims=True)
    idx1 = jnp.min(jnp.where(gates == gmax, col, E), axis=1)
    mask1 = col == idx1[:, None]
    lx = jnp.where(mask1, -jnp.inf, logits)
    lmax2 = jnp.max(lx, axis=1, keepdims=True)
    idx2 = jnp.min(jnp.where(lx == lmax2, col, E), axis=1)
    mask2 = col == idx2[:, None]
    m1f = mask1.astype(jnp.float32)
    m2f = mask2.astype(jnp.float32)

    tri = (lax.broadcasted_iota(jnp.int32, (S, S), 0)
           >= lax.broadcasted_iota(jnp.int32, (S, S), 1)).astype(jnp.bfloat16)
    cs1 = jnp.dot(tri, m1f.astype(jnp.bfloat16),
                  preferred_element_type=jnp.float32)
    cs2 = jnp.dot(tri, m2f.astype(jnp.bfloat16),
                  preferred_element_type=jnp.float32)
    loc1 = cs1 - 1.0
    loc2 = cs2 - 1.0 + jnp.sum(m1f, axis=0, keepdims=True)
    keep1 = m1f * (loc1 < C).astype(jnp.float32)
    keep2 = m2f * (loc2 < C).astype(jnp.float32)
    loc1s = jnp.sum(loc1 * keep1, axis=1).astype(jnp.int32)
    loc2s = jnp.sum(loc2 * keep2, axis=1).astype(jnp.int32)
    g1s = jnp.sum(gates * keep1, axis=1)
    g2s = jnp.sum(gates * keep2, axis=1)
    denom = jnp.maximum(g1s + g2s, jnp.float32(1.1920929e-07))
    g1n = g1s / denom
    g2n = g2s / denom
    k1 = jnp.sum(keep1, axis=1) > 0.0
    k2 = jnp.sum(keep2, axis=1) > 0.0
    slot1 = idx1 * C + loc1s
    slot2 = idx2 * C + loc2s
    sd1_ref[...] = jnp.where(k1, slot1, EC)
    sd2_ref[...] = jnp.where(k2, slot2, EC)
    sc1_ref[...] = jnp.where(k1, slot1, 0)
    sc2_ref[...] = jnp.where(k2, slot2, 0)
    zl = jnp.zeros((S, NL), jnp.float32)
    g1_ref[...] = jnp.where(k1, g1n, 0.0)[:, None] + zl
    g2_ref[...] = jnp.where(k2, g2n, 0.0)[:, None] + zl


def _gate_call(logits):
    return pl.pallas_call(
        _gate_kern,
        out_shape=[
            jax.ShapeDtypeStruct((S,), jnp.int32),
            jax.ShapeDtypeStruct((S,), jnp.int32),
            jax.ShapeDtypeStruct((S,), jnp.int32),
            jax.ShapeDtypeStruct((S,), jnp.int32),
            jax.ShapeDtypeStruct((S, NL), jnp.float32),
            jax.ShapeDtypeStruct((S, NL), jnp.float32),
        ],
    )(logits)


def _ffn_kern(x_ref, w1_ref, b1_ref, w2_ref, b2_ref, o_ref):
    f = pl.program_id(1)
    xb = x_ref[0].astype(jnp.bfloat16)
    h1 = jnp.dot(xb, w1_ref[0].astype(jnp.bfloat16),
                 preferred_element_type=jnp.float32)
    h1 = jax.nn.gelu(h1 + b1_ref[0])
    part = jnp.dot(h1.astype(jnp.bfloat16), w2_ref[0].astype(jnp.bfloat16),
                   preferred_element_type=jnp.float32)

    @pl.when(f == 0)
    def _():
        o_ref[0] = part + b2_ref[0]

    @pl.when(f > 0)
    def _():
        o_ref[0] += part


def _ffn_call(disp3, w1, b1, w2, b2):
    bf = 512
    return pl.pallas_call(
        _ffn_kern,
        grid=(E, DFF // bf),
        in_specs=[
            pl.BlockSpec((1, C, D), lambda e, f: (e, 0, 0)),
            pl.BlockSpec((1, D, bf), lambda e, f: (e, 0, f)),
            pl.BlockSpec((1, 1, bf), lambda e, f: (e, 0, f)),
            pl.BlockSpec((1, bf, D), lambda e, f: (e, f, 0)),
            pl.BlockSpec((1, 1, D), lambda e, f: (e, 0, 0)),
        ],
        out_specs=pl.BlockSpec((1, C, D), lambda e, f: (e, 0, 0)),
        out_shape=jax.ShapeDtypeStruct((E, C, D), jnp.float32),
    )(disp3, w1, b1.reshape(E, 1, DFF), w2, b2.reshape(E, 1, D))


@functools.cache
def _make_dispatch():
  mesh = plsc.VectorSubcoreMesh(core_axis_name="c", subcore_axis_name="s")

  @functools.partial(
      pl.kernel,
      mesh=mesh,
      out_type=jax.ShapeDtypeStruct((EC, D), jnp.float32),
      compiler_params=pltpu.CompilerParams(needs_layout_passes=False),
      scratch_types=[
          pltpu.VMEM((SRC_N,), jnp.int32),
          pltpu.VMEM((S,), jnp.int32),
          pltpu.VMEM((S,), jnp.int32),
          pltpu.VMEM((64, D), jnp.float32),
          pltpu.SemaphoreType.DMA,
      ],
  )
  def _dispatch_sc(sd1_hbm, sd2_hbm, tokpad_hbm, disp_hbm,
                   src_v, s1_v, s2_v, rows_v, sem):
    wid = lax.axis_index("s") * NC + lax.axis_index("c")
    pltpu.sync_copy(sd1_hbm, s1_v)
    pltpu.sync_copy(sd2_hbm, s2_v)

    fill = jnp.full((NL,), S, jnp.int32)

    def init_body(i, c):
      src_v[pl.ds(i * NL, NL)] = fill
      return c

    lax.fori_loop(0, SRC_N // NL, init_body, 0)

    def scat_body(i, c):
      ids = lax.broadcasted_iota(jnp.int32, (NL,), 0) + i * NL
      plsc.store_scatter(src_v, [s1_v[pl.ds(i * NL, NL)]], ids)
      plsc.store_scatter(src_v, [s2_v[pl.ds(i * NL, NL)]], ids)
      return c

    lax.fori_loop(0, S // NL, scat_body, 0)

    base = wid * SLOTS_PER_W

    def gath_body(j, c):
      off = base + j * 64
      pltpu.async_copy(tokpad_hbm.at[src_v.at[pl.ds(off, 64)]],
                       rows_v, sem).wait()
      pltpu.sync_copy(rows_v, disp_hbm.at[pl.ds(off, 64)])
      return c

    lax.fori_loop(0, SLOTS_PER_W // 64, gath_body, 0)

  return _dispatch_sc


@functools.cache
def _make_combine():
  mesh = plsc.VectorSubcoreMesh(core_axis_name="c", subcore_axis_name="s")

  @functools.partial(
      pl.kernel,
      mesh=mesh,
      out_type=jax.ShapeDtypeStruct((S, D), jnp.float32),
      scratch_types=[
          pltpu.VMEM((TOK_PER_W,), jnp.int32),
          pltpu.VMEM((TOK_PER_W,), jnp.int32),
          pltpu.VMEM((TOK_PER_W, NL), jnp.float32),
          pltpu.VMEM((TOK_PER_W, NL), jnp.float32),
          pltpu.VMEM((32, D), jnp.float32),
          pltpu.VMEM((32, D), jnp.float32),
          pltpu.VMEM((32, D), jnp.float32),
          pltpu.SemaphoreType.DMA,
          pltpu.SemaphoreType.DMA,
      ],
  )
  def _combine_sc(h2_hbm, y_hbm, sc1_hbm, sc2_hbm, g1_hbm, g2_hbm, out_hbm,
                  i1_v, i2_v, g1_v, g2_v, r1_v, r2_v, y_v, sem1, sem2):
    wid = lax.axis_index("s") * NC + lax.axis_index("c")
    tbase = wid * TOK_PER_W
    pltpu.sync_copy(sc1_hbm.at[pl.ds(tbase, TOK_PER_W)], i1_v)
    pltpu.sync_copy(sc2_hbm.at[pl.ds(tbase, TOK_PER_W)], i2_v)
    pltpu.sync_copy(g1_hbm.at[pl.ds(tbase, TOK_PER_W)], g1_v)
    pltpu.sync_copy(g2_hbm.at[pl.ds(tbase, TOK_PER_W)], g2_v)

    for half in range(TOK_PER_W // 32):
      t0 = tbase + half * 32
      cp1 = pltpu.async_copy(h2_hbm.at[i1_v.at[pl.ds(half * 32, 32)]],
                             r1_v, sem1)
      cp2 = pltpu.async_copy(h2_hbm.at[i2_v.at[pl.ds(half * 32, 32)]],
                             r2_v, sem2)
      pltpu.sync_copy(y_hbm.at[pl.ds(t0, 32)], y_v)
      cp1.wait()
      cp2.wait()

      def tok_body(t, c):
        a = g1_v[half * 32 + t, :]
        b = g2_v[half * 32 + t, :]

        def j_body(j, cc):
          sl = pl.ds(j * NL, NL)
          y_v[t, sl] = y_v[t, sl] + a * r1_v[t, sl] + b * r2_v[t, sl]
          return cc

        lax.fori_loop(0, D // NL, j_body, 0)
        return c

      lax.fori_loop(0, 32, tok_body, 0)
      pltpu.sync_copy(y_v, out_hbm.at[pl.ds(t0, 32)])

  return _combine_sc


def kernel(x, wq, wk, wv, wo, ln1_g, ln1_b, ln2_g, ln2_b, wg, w1, b1, w2, b2):
    x2 = x.reshape(S, D)
    qkv = _qkv_call(x2, wq, wk, wv, ln1_g.reshape(1, D), ln1_b.reshape(1, D))
    q4 = qkv[:, :D].reshape(S, H, DH).transpose(1, 0, 2)
    k4 = qkv[:, D:2 * D].reshape(S, H, DH).transpose(1, 0, 2)
    v4 = qkv[:, 2 * D:].reshape(S, H, DH).transpose(1, 0, 2)
    ctx4 = _attn_call(q4, k4, v4)
    ctx2 = ctx4.transpose(1, 0, 2).reshape(S, D)
    y, tokpad, logits = _post_call(ctx2, x2, wo, ln2_g.reshape(1, D),
                                   ln2_b.reshape(1, D), wg)
    sd1, sd2, sc1, sc2, g1, g2 = _gate_call(logits)
    disp = _make_dispatch()(sd1, sd2, tokpad)
    h2 = _ffn_call(disp.reshape(E, C, D), w1, b1, w2, b2)
    out = _make_combine()(h2.reshape(EC, D), y, sc1, sc2, g1, g2)
    return out.reshape(1, S, D)
